# baseline (speedup 1.0000x reference)
import functools

import jax
import jax.numpy as jnp
import numpy as np
from jax.experimental import pallas as pl
from jax.experimental.pallas import tpu as pltpu

_LANE = 128
_ACTION_DIM = 6


def _actor_kernel(x_ref, w1_ref, b1_ref, w2_ref, b2_ref, w3_ref, b3_ref,
                  mean_ref, std_ref, *, action_dim):
    x = x_ref[...].astype(jnp.bfloat16)
    h1 = jnp.maximum(
        jnp.dot(x, w1_ref[...], preferred_element_type=jnp.float32)
        + b1_ref[...], 0.0)
    h2 = jnp.maximum(
        jnp.dot(h1.astype(jnp.bfloat16), w2_ref[...],
                preferred_element_type=jnp.float32)
        + b2_ref[...], 0.0)
    raw = jnp.dot(h2.astype(jnp.bfloat16), w3_ref[...],
                  preferred_element_type=jnp.float32) + b3_ref[...]
    a = action_dim
    mean_ref[...] = jnp.clip(raw[:, :a], -100.0, 100.0)
    std_ref[...] = jnp.clip(
        jnp.exp(jnp.clip(raw[:, a:2 * a], -20.0, 2.0)), 0.01, 100.0)


def _pick_tile(batch):
    for tb in (4096, 2048, 1024, 512, 256, 128, 64, 32, 16, 8):
        if batch % tb == 0 and batch // tb >= 2:
            return tb
    return batch


def kernel(state, w1, b1, w2, b2, w3t, b3t):
    B, K = state.shape
    H = w1.shape[1]
    R = w3t.shape[0]
    A = _ACTION_DIM

    w1b = w1.astype(jnp.bfloat16)
    w2b = w2.astype(jnp.bfloat16)
    w3b = jnp.transpose(w3t).astype(jnp.bfloat16)
    b3 = jnp.transpose(b3t)

    TB = _pick_tile(B)
    n_tiles = B // TB

    def resident(arr):
        return pl.BlockSpec(arr.shape, lambda i: (0,) * arr.ndim)

    in_specs = [
        pl.BlockSpec((TB, K), lambda i: (i, 0)),
        resident(w1b), resident(b1),
        resident(w2b), resident(b2),
        resident(w3b), resident(b3),
    ]
    out_specs = [
        pl.BlockSpec((TB, A), lambda i: (i, 0)),
        pl.BlockSpec((TB, A), lambda i: (i, 0)),
    ]

    param_bytes = sum(int(np.prod(p.shape)) * p.dtype.itemsize
                      for p in (w1b, b1, w2b, b2, w3b, b3))
    cost = pl.CostEstimate(
        flops=2 * B * (K * H + H * H + H * R),
        transcendentals=B * A,
        bytes_accessed=4 * (B * K + 2 * B * A) + param_bytes,
    )

    mean, std = pl.pallas_call(
        functools.partial(_actor_kernel, action_dim=A),
        out_shape=[jax.ShapeDtypeStruct((B, A), jnp.float32),
                   jax.ShapeDtypeStruct((B, A), jnp.float32)],
        grid=(n_tiles,),
        in_specs=in_specs,
        out_specs=out_specs,
        compiler_params=pltpu.CompilerParams(
            dimension_semantics=("parallel",)),
        cost_estimate=cost,
    )(state, w1b, b1, w2b, b2, w3b, b3)
    return mean, std
